# Initial kernel scaffold; baseline (speedup 1.0000x reference)
import jax
import jax.numpy as jnp
from jax.experimental import pallas as pl


def kernel(x, edge_index, enc_W1, enc_b1, enc_W2, enc_b2, enc_Wl, enc_bl, enc_ln_w, enc_ln_b, dec_W1, dec_b1, dec_W2, dec_b2, dec_Wl, dec_bl, dec_ln_w, dec_ln_b, bn_w, bn_b, lin_W):
    raise NotImplementedError("write your pallas kernel here")


# speedup vs baseline: 8.8406x; 8.8406x over previous
import functools

import jax
import jax.numpy as jnp
from jax import lax
from jax.experimental import pallas as pl
from jax.experimental.pallas import tpu as pltpu
from jax.experimental.pallas import tpu_sc as plsc

_N = 10000
_E = 320000
_D = 128
_NC = 2
_NS = 16
_CHUNK = 80
_RSUB = 624
_RLAST = _N - _RSUB * (_NS - 1)

_mesh = plsc.VectorSubcoreMesh(core_axis_name="c", subcore_axis_name="s")


@functools.partial(
    pl.kernel,
    out_type=jax.ShapeDtypeStruct((_NC, _N), jnp.float32),
    mesh=_mesh,
    scratch_types=[
        pltpu.VMEM((_CHUNK,), jnp.int32),
        pltpu.VMEM((_CHUNK,), jnp.float32),
        pltpu.VMEM_SHARED((_N,), jnp.float32),
    ],
)
def _sc_degree(dst_hbm, zeros_hbm, out_hbm, didx, ones_v, acc):
    cid = lax.axis_index("c")
    sid = lax.axis_index("s")
    wid = sid * _NC + cid

    @pl.when(sid == 0)
    def _():
        pltpu.sync_copy(zeros_hbm, acc)

    for i in range(_CHUNK // 16):
        ones_v[pl.ds(16 * i, 16)] = jnp.ones((16,), jnp.float32)
    plsc.subcore_barrier()

    epw = _E // (_NC * _NS)

    def step(g, carry):
        base = pl.multiple_of(wid * epw + g * _CHUNK, 8)
        pltpu.sync_copy(dst_hbm.at[pl.ds(base, _CHUNK)], didx)
        pltpu.sync_copy(ones_v, acc.at[didx], add=True)
        return carry

    lax.fori_loop(0, epw // _CHUNK, step, 0)
    plsc.subcore_barrier()

    @pl.when(sid == 0)
    def _():
        pltpu.sync_copy(acc, out_hbm.at[cid])


@functools.partial(
    pl.kernel,
    out_type=jax.ShapeDtypeStruct((_NC, _N, _D), jnp.float32),
    mesh=_mesh,
    scratch_types=[
        pltpu.VMEM((_CHUNK,), jnp.int32),
        pltpu.VMEM((_CHUNK,), jnp.int32),
        pltpu.VMEM((_CHUNK, _D), jnp.float32),
        pltpu.VMEM_SHARED((_N, _D), jnp.float32),
        pltpu.SemaphoreType.DMA,
    ],
)
def _sc_mp1(table_hbm, src_hbm, dst_hbm, zeros_hbm, out_hbm,
            sidx, didx, rows, acc, sem):
    cid = lax.axis_index("c")
    sid = lax.axis_index("s")
    wid = sid * _NC + cid
    lo = pl.multiple_of(sid * _RSUB, 8)

    @pl.when(sid < _NS - 1)
    def _():
        pltpu.sync_copy(zeros_hbm.at[pl.ds(0, _RSUB)], acc.at[pl.ds(lo, _RSUB)])

    @pl.when(sid == _NS - 1)
    def _():
        pltpu.sync_copy(zeros_hbm, acc.at[pl.ds(_N - _RLAST, _RLAST)])

    plsc.subcore_barrier()

    epw = _E // (_NC * _NS)

    def step(g, carry):
        base = pl.multiple_of(wid * epw + g * _CHUNK, 8)
        pltpu.sync_copy(src_hbm.at[pl.ds(base, _CHUNK)], sidx)
        pltpu.async_copy(table_hbm.at[sidx], rows, sem).wait()
        pltpu.sync_copy(dst_hbm.at[pl.ds(base, _CHUNK)], didx)
        pltpu.sync_copy(rows, acc.at[didx], add=True)
        return carry

    lax.fori_loop(0, epw // _CHUNK, step, 0)
    plsc.subcore_barrier()

    @pl.when(sid < _NS - 1)
    def _():
        pltpu.sync_copy(acc.at[pl.ds(lo, _RSUB)],
                        out_hbm.at[cid, pl.ds(lo, _RSUB)])

    @pl.when(sid == _NS - 1)
    def _():
        pltpu.sync_copy(acc.at[pl.ds(_N - _RLAST, _RLAST)],
                        out_hbm.at[cid, pl.ds(_N - _RLAST, _RLAST)])


@functools.partial(
    pl.kernel,
    out_type=jax.ShapeDtypeStruct((_NC, _N, _D), jnp.float32),
    mesh=_mesh,
    scratch_types=[
        pltpu.VMEM((_CHUNK,), jnp.int32),
        pltpu.VMEM((_CHUNK,), jnp.int32),
        pltpu.VMEM((_CHUNK, _D), jnp.float32),
        pltpu.VMEM_SHARED((_N, _D), jnp.float32),
        pltpu.SemaphoreType.DMA,
    ],
)
def _sc_mp2(table_e_hbm, table_d_hbm, src_hbm, dst_hbm, zeros_hbm, out_hbm,
            sidx, didx, rows, acc, sem):
    cid = lax.axis_index("c")
    sid = lax.axis_index("s")
    lo = pl.multiple_of(sid * _RSUB, 8)

    @pl.when(sid < _NS - 1)
    def _():
        pltpu.sync_copy(zeros_hbm.at[pl.ds(0, _RSUB)], acc.at[pl.ds(lo, _RSUB)])

    @pl.when(sid == _NS - 1)
    def _():
        pltpu.sync_copy(zeros_hbm, acc.at[pl.ds(_N - _RLAST, _RLAST)])

    plsc.subcore_barrier()

    epw = _E // _NS

    def make_step(table_hbm):
        def step(g, carry):
            base = pl.multiple_of(sid * epw + g * _CHUNK, 8)
            pltpu.sync_copy(src_hbm.at[pl.ds(base, _CHUNK)], sidx)
            pltpu.async_copy(table_hbm.at[sidx], rows, sem).wait()
            pltpu.sync_copy(dst_hbm.at[pl.ds(base, _CHUNK)], didx)
            pltpu.sync_copy(rows, acc.at[didx], add=True)
            return carry
        return step

    @pl.when(cid == 0)
    def _():
        lax.fori_loop(0, epw // _CHUNK, make_step(table_e_hbm), 0)

    @pl.when(cid == 1)
    def _():
        lax.fori_loop(0, epw // _CHUNK, make_step(table_d_hbm), 0)

    plsc.subcore_barrier()

    @pl.when(sid < _NS - 1)
    def _():
        pltpu.sync_copy(acc.at[pl.ds(lo, _RSUB)],
                        out_hbm.at[cid, pl.ds(lo, _RSUB)])

    @pl.when(sid == _NS - 1)
    def _():
        pltpu.sync_copy(acc.at[pl.ds(_N - _RLAST, _RLAST)],
                        out_hbm.at[cid, pl.ds(_N - _RLAST, _RLAST)])


_BLK = 1000
_NBLK = _N // _BLK


def _tc_prep_body(deg2t_ref, x_ref, dinv_ref, xs_ref):
    d2 = deg2t_ref[...]
    total = d2[:, 0:1] + d2[:, 1:2]
    dcol = jnp.where(total > 0, lax.rsqrt(total), 0.0)
    dinv_ref[...] = dcol
    xs_ref[...] = x_ref[...] * dcol


def _tc_prep(deg2t, x):
    return pl.pallas_call(
        _tc_prep_body,
        grid=(_NBLK,),
        in_specs=[
            pl.BlockSpec((_BLK, _NC), lambda i: (i, 0)),
            pl.BlockSpec((_BLK, _D), lambda i: (i, 0)),
        ],
        out_specs=[
            pl.BlockSpec((_BLK, 1), lambda i: (i, 0)),
            pl.BlockSpec((_BLK, _D), lambda i: (i, 0)),
        ],
        out_shape=[
            jax.ShapeDtypeStruct((_N, 1), jnp.float32),
            jax.ShapeDtypeStruct((_N, _D), jnp.float32),
        ],
    )(deg2t, x)


def _tc_mid_body(agg_ref, dinv_ref, w1et_ref, b1e_ref, w1dt_ref, b1d_ref,
                 h1e_ref, h1d_ref, ht_ref):
    dcol = dinv_ref[...]
    xa = (agg_ref[0] + agg_ref[1]) * dcol
    h1e = jnp.tanh(jnp.dot(xa, w1et_ref[...],
                           preferred_element_type=jnp.float32, precision=lax.Precision.HIGHEST) + b1e_ref[...])
    h1d = jnp.tanh(jnp.dot(xa, w1dt_ref[...],
                           preferred_element_type=jnp.float32, precision=lax.Precision.HIGHEST) + b1d_ref[...])
    h1e_ref[...] = h1e
    h1d_ref[...] = h1d
    ht_ref[0] = h1e * dcol
    ht_ref[1] = h1d * dcol


def _tc_mid(agg0, dinv, w1et, b1e, w1dt, b1d):
    wspec = pl.BlockSpec((_D, _D), lambda i: (0, 0))
    bspec = pl.BlockSpec((1, _D), lambda i: (0, 0))
    return pl.pallas_call(
        _tc_mid_body,
        grid=(_NBLK,),
        in_specs=[
            pl.BlockSpec((_NC, _BLK, _D), lambda i: (0, i, 0)),
            pl.BlockSpec((_BLK, 1), lambda i: (i, 0)),
            wspec, bspec, wspec, bspec,
        ],
        out_specs=[
            pl.BlockSpec((_BLK, _D), lambda i: (i, 0)),
            pl.BlockSpec((_BLK, _D), lambda i: (i, 0)),
            pl.BlockSpec((_NC, _BLK, _D), lambda i: (0, i, 0)),
        ],
        out_shape=[
            jax.ShapeDtypeStruct((_N, _D), jnp.float32),
            jax.ShapeDtypeStruct((_N, _D), jnp.float32),
            jax.ShapeDtypeStruct((_NC, _N, _D), jnp.float32),
        ],
    )(agg0, dinv, w1et, b1e, w1dt, b1d)


def _ln(v, w, b):
    mu = jnp.mean(v, axis=-1, keepdims=True)
    var = jnp.mean((v - mu) * (v - mu), axis=-1, keepdims=True)
    return (v - mu) * lax.rsqrt(var + 1e-5) * w + b


def _tc_fin_body(agg_ref, dinv_ref, h1e_ref, h1d_ref,
                 w2et_ref, b2e_ref, w2dt_ref, b2d_ref,
                 wlet_ref, ble_ref, wldt_ref, bld_ref,
                 lnwe_ref, lnbe_ref, lnwd_ref, lnbd_ref,
                 ne_ref, sums_ref, sumsq_ref):
    dcol = dinv_ref[...]
    xae = agg_ref[0] * dcol
    xad = agg_ref[1] * dcol
    h2e = jnp.tanh(jnp.dot(xae, w2et_ref[...],
                           preferred_element_type=jnp.float32, precision=lax.Precision.HIGHEST) + b2e_ref[...])
    h2d = jnp.tanh(jnp.dot(xad, w2dt_ref[...],
                           preferred_element_type=jnp.float32, precision=lax.Precision.HIGHEST) + b2d_ref[...])
    jke = jnp.maximum(h1e_ref[...], h2e)
    jkd = jnp.maximum(h1d_ref[...], h2d)
    oe = jnp.dot(jke, wlet_ref[...], preferred_element_type=jnp.float32, precision=lax.Precision.HIGHEST) + ble_ref[...]
    od = jnp.dot(jkd, wldt_ref[...], preferred_element_type=jnp.float32, precision=lax.Precision.HIGHEST) + bld_ref[...]
    oe = _ln(oe, lnwe_ref[...], lnbe_ref[...])
    od = _ln(od, lnwd_ref[...], lnbd_ref[...])
    ne = (oe + od) * 0.5
    ne_ref[...] = ne
    sums_ref[0] = jnp.sum(ne, axis=0, keepdims=True)
    sumsq_ref[0] = jnp.sum(ne * ne, axis=0, keepdims=True)


def _tc_fin(agg1, dinv, h1e, h1d, w2et, b2e, w2dt, b2d,
            wlet, ble, wldt, bld, lnwe, lnbe, lnwd, lnbd):
    wspec = pl.BlockSpec((_D, _D), lambda i: (0, 0))
    bspec = pl.BlockSpec((1, _D), lambda i: (0, 0))
    return pl.pallas_call(
        _tc_fin_body,
        grid=(_NBLK,),
        in_specs=[
            pl.BlockSpec((_NC, _BLK, _D), lambda i: (0, i, 0)),
            pl.BlockSpec((_BLK, 1), lambda i: (i, 0)),
            pl.BlockSpec((_BLK, _D), lambda i: (i, 0)),
            pl.BlockSpec((_BLK, _D), lambda i: (i, 0)),
            wspec, bspec, wspec, bspec,
            wspec, bspec, wspec, bspec,
            bspec, bspec, bspec, bspec,
        ],
        out_specs=[
            pl.BlockSpec((_BLK, _D), lambda i: (i, 0)),
            pl.BlockSpec((1, 1, _D), lambda i: (i, 0, 0)),
            pl.BlockSpec((1, 1, _D), lambda i: (i, 0, 0)),
        ],
        out_shape=[
            jax.ShapeDtypeStruct((_N, _D), jnp.float32),
            jax.ShapeDtypeStruct((_NBLK, 1, _D), jnp.float32),
            jax.ShapeDtypeStruct((_NBLK, 1, _D), jnp.float32),
        ],
    )(agg1, dinv, h1e, h1d, w2et, b2e, w2dt, b2d,
      wlet, ble, wldt, bld, lnwe, lnbe, lnwd, lnbd)


def _tc_head_body(ne_ref, sums_ref, sumsq_ref, bnw_ref, bnb_ref, lin_ref,
                  out_ref):
    n = jnp.float32(_N)
    mu = jnp.sum(sums_ref[:, 0, :], axis=0) / n
    q = jnp.sum(sumsq_ref[:, 0, :], axis=0) / n
    var = q - mu * mu
    inv = lax.rsqrt(var + 1e-5)
    wp = lin_ref[0, :] * bnw_ref[0, :] * inv
    c = jnp.sum(lin_ref[0, :] * (bnb_ref[0, :] - mu * inv * bnw_ref[0, :]))
    out_ref[...] = jnp.dot(ne_ref[...], wp[:, None],
                           preferred_element_type=jnp.float32, precision=lax.Precision.HIGHEST) + c


def _tc_head(ne, sums, sumsq, bnw, bnb, lin):
    bspec = pl.BlockSpec((1, _D), lambda i: (0, 0))
    return pl.pallas_call(
        _tc_head_body,
        grid=(_NBLK,),
        in_specs=[
            pl.BlockSpec((_BLK, _D), lambda i: (i, 0)),
            pl.BlockSpec((_NBLK, 1, _D), lambda i: (0, 0, 0)),
            pl.BlockSpec((_NBLK, 1, _D), lambda i: (0, 0, 0)),
            bspec, bspec, bspec,
        ],
        out_specs=pl.BlockSpec((_BLK, 1), lambda i: (i, 0)),
        out_shape=jax.ShapeDtypeStruct((_N, 1), jnp.float32),
    )(ne, sums, sumsq, bnw, bnb, lin)


def kernel(x, edge_index, enc_W1, enc_b1, enc_W2, enc_b2, enc_Wl, enc_bl,
           enc_ln_w, enc_ln_b, dec_W1, dec_b1, dec_W2, dec_b2, dec_Wl, dec_bl,
           dec_ln_w, dec_ln_b, bn_w, bn_b, lin_W):
    src = edge_index[0]
    dst = edge_index[1]
    zeros_deg = jnp.zeros((_N,), jnp.float32)
    zeros_rows = jnp.zeros((_RLAST, _D), jnp.float32)

    deg2 = _sc_degree(dst, zeros_deg)
    dinv, xs = _tc_prep(deg2.T, x)

    agg0 = _sc_mp1(xs, src, dst, zeros_rows)

    row = lambda v: v.reshape(1, _D)
    h1e, h1d, ht = _tc_mid(agg0, dinv, enc_W1.T, row(enc_b1),
                           dec_W1.T, row(dec_b1))

    agg1 = _sc_mp2(ht[0], ht[1], src, dst, zeros_rows)

    ne, sums, sumsq = _tc_fin(
        agg1, dinv, h1e, h1d, enc_W2.T, row(enc_b2), dec_W2.T, row(dec_b2),
        enc_Wl.T, row(enc_bl), dec_Wl.T, row(dec_bl),
        row(enc_ln_w), row(enc_ln_b), row(dec_ln_w), row(dec_ln_b))

    out = _tc_head(ne, sums, sumsq, row(bn_w), row(bn_b), lin_W)
    return out[:, 0]
